# Initial kernel scaffold; baseline (speedup 1.0000x reference)
import jax
import jax.numpy as jnp
from jax.experimental import pallas as pl


def kernel(queries, keys, similarity_weights, W_struct, b_struct, W_sem, b_sem, W_stat, b_stat, W_cont, b_cont, W_a1, b_a1, W_a2, b_a2, W_u1, b_u1, W_u2, b_u2):
    raise NotImplementedError("write your pallas kernel here")


# speedup vs baseline: 7.6164x; 7.6164x over previous
import functools

import jax
import jax.numpy as jnp
from jax import lax
from jax.experimental import pallas as pl
from jax.experimental.pallas import tpu as pltpu
from jax.experimental.pallas import tpu_sc as plsc

TOPK = 50
TEMP = 0.1
KB = 1024
BLK = 128
QT = 128
NEG = -1e30
IBIG = 2**30


def _mm(a, b):
    return lax.dot_general(a, b, (((1,), (0,)), ((), ())),
                           preferred_element_type=jnp.float32,
                           precision=lax.Precision.HIGHEST)


def _normalize(y):
    n = jnp.sqrt(jnp.sum(y * y, axis=-1, keepdims=True))
    return y / (n + 1e-12)


def _a1_body(qt_ref, kt_ref, sim_ref, m_ref, *, K, nsub):
    i = pl.program_id(0)
    s = lax.dot_general(qt_ref[...], kt_ref[...], (((1,), (1,)), ((), ())),
                        preferred_element_type=jnp.float32)
    s = s / jnp.float32(TEMP)
    col = i * KB + lax.broadcasted_iota(jnp.int32, s.shape, 1)
    s = jnp.where(col < K, s, NEG)
    sim_ref[...] = s
    sq = s.reshape(s.shape[0], nsub, BLK)
    m_ref[...] = jnp.max(sq, axis=2)[None]


def _phase_a1(qt_bf, kt_bf, K):
    Q, D = qt_bf.shape
    K_pad = kt_bf.shape[0]
    nstep = K_pad // KB
    nsub = KB // BLK
    body = functools.partial(_a1_body, K=K, nsub=nsub)
    sim, m3 = pl.pallas_call(
        body,
        grid=(nstep,),
        in_specs=[
            pl.BlockSpec((Q, D), lambda i: (0, 0)),
            pl.BlockSpec((KB, D), lambda i: (i, 0)),
        ],
        out_specs=[
            pl.BlockSpec((Q, KB), lambda i: (0, i)),
            pl.BlockSpec((1, Q, nsub), lambda i: (i, 0, 0)),
        ],
        out_shape=[
            jax.ShapeDtypeStruct((Q, K_pad), jnp.float32),
            jax.ShapeDtypeStruct((nstep, Q, nsub), jnp.float32),
        ],
    )(qt_bf, kt_bf)
    return sim, m3


def _select_topk(src_ref, key_ref_or_none, val_ref, key_out_ref,
                 v_s, av_s, ak_s):
    v0 = src_ref[...]
    R, W = v0.shape
    if key_ref_or_none is None:
        keys_c = lax.broadcasted_iota(jnp.int32, (R, W), 1)
    else:
        keys_c = key_ref_or_none
    v_s[...] = v0
    av_s[...] = jnp.zeros((R, TOPK), jnp.float32)
    ak_s[...] = jnp.zeros((R, TOPK), jnp.int32)
    tlane = lax.broadcasted_iota(jnp.int32, (R, TOPK), 1)

    def it(t, _):
        v = v_s[...]
        mx = jnp.max(v, axis=1, keepdims=True)
        kid = jnp.min(jnp.where(v == mx, keys_c, IBIG), axis=1, keepdims=True)
        sel = tlane == t
        av_s[...] = jnp.where(sel, mx, av_s[...])
        ak_s[...] = jnp.where(sel, kid, ak_s[...])
        v_s[...] = jnp.where(keys_c == kid, NEG, v)
        return 0

    lax.fori_loop(0, TOPK, it, 0)
    val_ref[...] = av_s[...]
    key_out_ref[...] = ak_s[...]


def _a2_body(m_ref, mval_ref, bid_ref, v_s, av_s, ak_s):
    _select_topk(m_ref, None, mval_ref, bid_ref, v_s, av_s, ak_s)


def _phase_a2(M):
    Q, NB = M.shape
    _, bids = pl.pallas_call(
        _a2_body,
        out_shape=[
            jax.ShapeDtypeStruct((Q, TOPK), jnp.float32),
            jax.ShapeDtypeStruct((Q, TOPK), jnp.int32),
        ],
        scratch_shapes=[
            pltpu.VMEM((Q, NB), jnp.float32),
            pltpu.VMEM((Q, TOPK), jnp.float32),
            pltpu.VMEM((Q, TOPK), jnp.int32),
        ],
    )(M)
    return bids


def _sc_gather_rows(table, idx2d):
    n_chunks, CH = idx2d.shape
    B = n_chunks * CH
    D = table.shape[1]
    info = plsc.get_sparse_core_info()
    NC, NS = info.num_cores, info.num_subcores
    NW = NC * NS
    cpw = n_chunks // NW
    mesh = plsc.VectorSubcoreMesh(core_axis_name="c", subcore_axis_name="s")

    @functools.partial(
        pl.kernel, mesh=mesh,
        out_type=jax.ShapeDtypeStruct((B, D), jnp.float32),
        scratch_types=[
            pltpu.VMEM((CH,), jnp.int32),
            pltpu.VMEM((CH, D), jnp.float32),
            pltpu.SemaphoreType.DMA,
        ],
    )
    def k(table_hbm, idx_hbm, out_hbm, idx_v, rows_v, sem):
        wid = lax.axis_index("s") * NC + lax.axis_index("c")

        def step(c, _):
            j = wid * cpw + c
            pltpu.sync_copy(idx_hbm.at[j], idx_v)
            pltpu.async_copy(table_hbm.at[idx_v], rows_v, sem).wait()
            pltpu.sync_copy(rows_v, out_hbm.at[pl.ds(j * CH, CH)])
            return _

        lax.fori_loop(0, cpw, step, 0)

    return k(table, idx2d)


def _d_body(cand_ref, bid_ref, val_ref, col_ref, v_s, cc_s, av_s, ak_s):
    bid = bid_ref[...]
    R, W = cc_s.shape
    sub = lax.broadcasted_iota(jnp.int32, (R, TOPK, BLK), 2)
    cc_s[...] = (bid[:, :, None] * BLK + sub).reshape(R, W)
    _select_topk(cand_ref, cc_s[...], val_ref, col_ref, v_s, av_s, ak_s)


def _phase_d(cand, bids):
    Q, W = cand.shape
    return pl.pallas_call(
        _d_body,
        grid=(Q // QT,),
        in_specs=[
            pl.BlockSpec((QT, W), lambda i: (i, 0)),
            pl.BlockSpec((QT, TOPK), lambda i: (i, 0)),
        ],
        out_specs=[
            pl.BlockSpec((QT, TOPK), lambda i: (i, 0)),
            pl.BlockSpec((QT, TOPK), lambda i: (i, 0)),
        ],
        out_shape=[
            jax.ShapeDtypeStruct((Q, TOPK), jnp.float32),
            jax.ShapeDtypeStruct((Q, TOPK), jnp.int32),
        ],
        scratch_shapes=[
            pltpu.VMEM((QT, W), jnp.float32),
            pltpu.VMEM((QT, W), jnp.int32),
            pltpu.VMEM((QT, TOPK), jnp.float32),
            pltpu.VMEM((QT, TOPK), jnp.int32),
        ],
    )(cand, bids)


def _f_body(q_ref, tv_ref, n_ref, wa1q_ref, wa1n_ref, ba1_ref, wa2_ref,
            ba2_ref, wu1q_ref, wu1r_ref, bu1_ref, wu2_ref, bu2_ref, out_ref):
    q = q_ref[...]
    tv = tv_ref[...]
    n3 = n_ref[...]
    D = q.shape[1]

    mx = jnp.max(tv, axis=1, keepdims=True)
    e = jnp.exp(tv - mx)
    rw = e / jnp.sum(e, axis=1, keepdims=True)
    retrieved = jnp.sum(rw[:, :, None] * n3, axis=1)

    qa = _mm(q, wa1q_ref[...]) + ba1_ref[...]
    nflat = n3.reshape(-1, D)
    nb = _mm(nflat, wa1n_ref[...])
    h = jnp.maximum(nb.reshape(n3.shape) + qa[:, None, :], 0.0)
    scores = jnp.sum(h * wa2_ref[...][None], axis=2) + ba2_ref[0, 0]
    smx = jnp.max(scores, axis=1, keepdims=True)
    ex = jnp.exp(scores - smx)
    attn = ex / jnp.sum(ex, axis=1, keepdims=True)
    agg = jnp.sum(attn[:, :, None] * n3, axis=1)

    h2 = jnp.maximum(_mm(q, wu1q_ref[...]) + _mm(retrieved, wu1r_ref[...])
                     + bu1_ref[...], 0.0)
    out_ref[...] = _mm(h2, wu2_ref[...]) + bu2_ref[...] + agg


def _phase_f(queries, top_vals, neigh3, Wa1q, Wa1n, ba1, wa2row, ba2,
             Wu1q, Wu1r, bu1, Wu2, bu2):
    Q, D = queries.shape
    cst = lambda *shape: pl.BlockSpec(shape, lambda i: (0,) * len(shape))
    return pl.pallas_call(
        _f_body,
        grid=(Q // QT,),
        in_specs=[
            pl.BlockSpec((QT, D), lambda i: (i, 0)),
            pl.BlockSpec((QT, TOPK), lambda i: (i, 0)),
            pl.BlockSpec((QT, TOPK, D), lambda i: (i, 0, 0)),
            cst(D, D), cst(D, D), cst(1, D), cst(1, D), cst(1, 1),
            cst(D, D), cst(D, D), cst(1, D), cst(D, D), cst(1, D),
        ],
        out_specs=pl.BlockSpec((QT, D), lambda i: (i, 0)),
        out_shape=jax.ShapeDtypeStruct((Q, D), jnp.float32),
    )(queries, top_vals, neigh3, Wa1q, Wa1n, ba1, wa2row, ba2,
      Wu1q, Wu1r, bu1, Wu2, bu2)


def kernel(queries, keys, similarity_weights, W_struct, b_struct, W_sem,
           b_sem, W_stat, b_stat, W_cont, b_cont, W_a1, b_a1, W_a2, b_a2,
           W_u1, b_u1, W_u2, b_u2):
    Q, D = queries.shape
    K = keys.shape[0]

    w = jax.nn.softmax(similarity_weights)

    def _tf(e):
        we = (w[0] * (e @ W_struct + b_struct)
              + w[1] * (e @ W_sem + b_sem)
              + w[2] * (e @ W_stat + b_stat)
              + w[3] * (e @ W_cont + b_cont))
        return we / (jnp.linalg.norm(we, axis=1, keepdims=True) + 1e-12)

    qt_bf = _tf(queries).astype(jnp.bfloat16)
    K_pad = ((K + KB - 1) // KB) * KB
    kt_bf = jnp.pad(_tf(keys).astype(jnp.bfloat16), ((0, K_pad - K), (0, 0)))
    NB = K_pad // BLK

    sim, m3 = _phase_a1(qt_bf, kt_bf, K)
    M = jnp.transpose(m3, (1, 0, 2)).reshape(Q, NB)
    bids = _phase_a2(M)

    gidx = (jnp.arange(Q, dtype=jnp.int32)[:, None] * NB + bids).reshape(-1)
    CH = 80
    cand_rows = _sc_gather_rows(sim.reshape(Q * NB, BLK),
                                gidx.reshape(-1, CH))
    cand = cand_rows.reshape(Q, TOPK * BLK)

    top_vals, top_cols = _phase_d(cand, bids)

    neigh = _sc_gather_rows(keys, top_cols.reshape(-1).reshape(-1, CH))
    neigh3 = neigh.reshape(Q, TOPK, D)

    out = _phase_f(
        queries, top_vals, neigh3,
        W_a1[:D], W_a1[D:], b_a1.reshape(1, D),
        W_a2.reshape(1, D), b_a2.reshape(1, 1),
        W_u1[:D], W_u1[D:], b_u1.reshape(1, D),
        W_u2, b_u2.reshape(1, D),
    )
    return out
